# baseline (speedup 1.0000x reference)
import jax
import jax.numpy as jnp
from jax import lax
from jax.experimental import pallas as pl
from jax.experimental.pallas import tpu as pltpu
from jax.experimental.pallas import tpu_sc as plsc

K = 64
D = 1024
N = 8192
NC = 2
NS = 16
NW = NC * NS
L = 16
DW = D // NW
CHUNK = 256
NCHUNK = N // CHUNK


def _body(cent_hbm, st_hbm, keys_hbm, vals_hbm, asg_hbm, cnt0_hbm,
          outc_hbm, outs_hbm, outn_hbm,
          acc_k, acc_v, cnt_acc,
          kbuf, vbuf, idxbuf, nrows, cbuf, sbuf, outcnt):
    cid = lax.axis_index("c")
    sid = lax.axis_index("s")
    w = cid * NS + sid
    col0 = w * DW

    zf16 = jnp.zeros((L,), jnp.float32)
    ones16 = jnp.ones((L,), jnp.float32)

    def _zf(r, _):
        acc_k[r, pl.ds(0, L)] = zf16
        acc_k[r, pl.ds(L, L)] = zf16
        acc_v[r, pl.ds(0, L)] = zf16
        acc_v[r, pl.ds(L, L)] = zf16
        cnt_acc[r, :] = zf16
        return 0
    lax.fori_loop(0, K, _zf, 0)

    def _chunk(nck, _):
        base = nck * CHUNK
        pltpu.sync_copy(asg_hbm.at[pl.ds(base, CHUNK)], idxbuf)
        pltpu.sync_copy(keys_hbm.at[pl.ds(base, CHUNK), pl.ds(col0, DW)], kbuf)
        pltpu.sync_copy(vals_hbm.at[pl.ds(base, CHUNK), pl.ds(col0, DW)], vbuf)

        def _grp(q, _):
            iv = idxbuf[pl.ds(q * L, L)]
            for t in range(L):
                a = iv[t]
                tok = q * L + t
                plsc.addupdate(acc_k.at[a, pl.ds(0, L)],
                               kbuf[tok, pl.ds(0, L)])
                plsc.addupdate(acc_k.at[a, pl.ds(L, L)],
                               kbuf[tok, pl.ds(L, L)])
                plsc.addupdate(acc_v.at[a, pl.ds(0, L)],
                               vbuf[tok, pl.ds(0, L)])
                plsc.addupdate(acc_v.at[a, pl.ds(L, L)],
                               vbuf[tok, pl.ds(L, L)])
                plsc.addupdate(cnt_acc.at[a], ones16)
            return 0

        lax.fori_loop(0, CHUNK // L, _grp, 0)
        return 0

    lax.fori_loop(0, NCHUNK, _chunk, 0)

    pltpu.sync_copy(cnt0_hbm, nrows)
    pltpu.sync_copy(cent_hbm.at[:, pl.ds(col0, DW)], cbuf)
    pltpu.sync_copy(st_hbm.at[:, pl.ds(col0, DW)], sbuf)

    def _mg(r, _):
        nvec = nrows[r, :]
        mvec = cnt_acc[r, :]
        denom = nvec + mvec
        pos = denom > 0.5
        inv = 1.0 / jnp.where(pos, denom, 1.0)
        for j in range(DW // L):
            sl = pl.ds(j * L, L)
            c = cbuf[r, sl]
            s = sbuf[r, sl]
            cbuf[r, sl] = jnp.where(pos, (nvec * c + acc_k[r, sl]) * inv, c)
            sbuf[r, sl] = jnp.where(pos, (nvec * s + acc_v[r, sl]) * inv, s)
        return 0

    lax.fori_loop(0, K, _mg, 0)
    pltpu.sync_copy(cbuf, outc_hbm.at[:, pl.ds(col0, DW)])
    pltpu.sync_copy(sbuf, outs_hbm.at[:, pl.ds(col0, DW)])

    @pl.when(w == 0)
    def _():
        lane = lax.iota(jnp.int32, L)
        for j in range(K // L):
            acc = zf16
            for r in range(L):
                row = cnt_acc[j * L + r, :] + nrows[j * L + r, :]
                acc = jnp.where(lane == r, row, acc)
            outcnt[pl.ds(j * L, L)] = acc.astype(jnp.int32)
        pltpu.sync_copy(outcnt, outn_hbm)


@jax.jit
def _run(centroids, states, keys, values, assign, cnt0):
    mesh = plsc.VectorSubcoreMesh(core_axis_name="c", subcore_axis_name="s")
    f = pl.kernel(
        _body,
        out_type=(
            jax.ShapeDtypeStruct((K, D), jnp.float32),
            jax.ShapeDtypeStruct((K, D), jnp.float32),
            jax.ShapeDtypeStruct((K,), jnp.int32),
        ),
        mesh=mesh,
        compiler_params=pltpu.CompilerParams(use_tc_tiling_on_sc=False),
        scratch_types=[
            pltpu.VMEM((K, DW), jnp.float32),
            pltpu.VMEM((K, DW), jnp.float32),
            pltpu.VMEM((K, L), jnp.float32),
            pltpu.VMEM((CHUNK, DW), jnp.float32),
            pltpu.VMEM((CHUNK, DW), jnp.float32),
            pltpu.VMEM((CHUNK,), jnp.int32),
            pltpu.VMEM((K, L), jnp.float32),
            pltpu.VMEM((K, DW), jnp.float32),
            pltpu.VMEM((K, DW), jnp.float32),
            pltpu.VMEM((K,), jnp.int32),
        ],
    )
    return f(centroids, states, keys, values, assign, cnt0)


def kernel(centroids, states, counts, keys, values, assign):
    assign = assign.astype(jnp.int32)
    cnt0 = jnp.broadcast_to(
        counts.astype(jnp.float32)[:, None], (K, L))
    return _run(centroids, states, keys, values, assign, cnt0)


# speedup vs baseline: 1.3568x; 1.3568x over previous
import jax
import jax.numpy as jnp
from jax import lax
from jax.experimental import pallas as pl
from jax.experimental.pallas import tpu as pltpu
from jax.experimental.pallas import tpu_sc as plsc

K = 64
D = 1024
N = 8192
NC = 2
NS = 16
NW = NC * NS
L = 16
DW = D // NW
CHUNK = 256
NCHUNK = N // CHUNK


def _body(cent_hbm, st_hbm, keys_hbm, vals_hbm, asg_hbm, cnt0_hbm,
          outc_hbm, outs_hbm, outn_hbm,
          acc_k, acc_v, cnt_acc,
          kbufa, vbufa, idxbufa, kbufb, vbufb, idxbufb,
          nrows, cbuf, sbuf, outcnt, sema, semb):
    cid = lax.axis_index("c")
    sid = lax.axis_index("s")
    w = cid * NS + sid
    col0 = w * DW

    zf16 = jnp.zeros((L,), jnp.float32)
    ones16 = jnp.ones((L,), jnp.float32)

    def _zf(r, _):
        acc_k[r, pl.ds(0, L)] = zf16
        acc_k[r, pl.ds(L, L)] = zf16
        acc_v[r, pl.ds(0, L)] = zf16
        acc_v[r, pl.ds(L, L)] = zf16
        cnt_acc[r, :] = zf16
        return 0
    lax.fori_loop(0, K, _zf, 0)

    def _start(nck, kb, vb, ib, sem):
        base = nck * CHUNK
        pltpu.async_copy(asg_hbm.at[pl.ds(base, CHUNK)], ib, sem)
        pltpu.async_copy(
            keys_hbm.at[pl.ds(base, CHUNK), pl.ds(col0, DW)], kb, sem)
        pltpu.async_copy(
            vals_hbm.at[pl.ds(base, CHUNK), pl.ds(col0, DW)], vb, sem)

    def _wait(kb, vb, ib, sem):
        pltpu.make_async_copy(asg_hbm.at[pl.ds(0, CHUNK)], ib, sem).wait()
        pltpu.make_async_copy(
            keys_hbm.at[pl.ds(0, CHUNK), pl.ds(0, DW)], kb, sem).wait()
        pltpu.make_async_copy(
            vals_hbm.at[pl.ds(0, CHUNK), pl.ds(0, DW)], vb, sem).wait()

    def _compute(kb, vb, ib):
        def _grp(q, _):
            iv = ib[pl.ds(q * L, L)]
            for t in range(L):
                a = iv[t]
                tok = q * L + t
                plsc.addupdate(acc_k.at[a, pl.ds(0, L)], kb[tok, pl.ds(0, L)])
                plsc.addupdate(acc_k.at[a, pl.ds(L, L)], kb[tok, pl.ds(L, L)])
                plsc.addupdate(acc_v.at[a, pl.ds(0, L)], vb[tok, pl.ds(0, L)])
                plsc.addupdate(acc_v.at[a, pl.ds(L, L)], vb[tok, pl.ds(L, L)])
                plsc.addupdate(cnt_acc.at[a], ones16)
            return 0

        lax.fori_loop(0, CHUNK // L, _grp, 0)

    _start(0, kbufa, vbufa, idxbufa, sema)

    def _pair(gg, _):
        _wait(kbufa, vbufa, idxbufa, sema)
        _start(2 * gg + 1, kbufb, vbufb, idxbufb, semb)
        _compute(kbufa, vbufa, idxbufa)
        _wait(kbufb, vbufb, idxbufb, semb)

        @pl.when(gg < NCHUNK // 2 - 1)
        def _():
            _start(2 * gg + 2, kbufa, vbufa, idxbufa, sema)

        _compute(kbufb, vbufb, idxbufb)
        return 0

    lax.fori_loop(0, NCHUNK // 2, _pair, 0)

    pltpu.sync_copy(cnt0_hbm, nrows)
    pltpu.sync_copy(cent_hbm.at[:, pl.ds(col0, DW)], cbuf)
    pltpu.sync_copy(st_hbm.at[:, pl.ds(col0, DW)], sbuf)

    def _mg(r, _):
        nvec = nrows[r, :]
        mvec = cnt_acc[r, :]
        denom = nvec + mvec
        pos = denom > 0.5
        inv = 1.0 / jnp.where(pos, denom, 1.0)
        for j in range(DW // L):
            sl = pl.ds(j * L, L)
            c = cbuf[r, sl]
            s = sbuf[r, sl]
            cbuf[r, sl] = jnp.where(pos, (nvec * c + acc_k[r, sl]) * inv, c)
            sbuf[r, sl] = jnp.where(pos, (nvec * s + acc_v[r, sl]) * inv, s)
        return 0

    lax.fori_loop(0, K, _mg, 0)
    pltpu.sync_copy(cbuf, outc_hbm.at[:, pl.ds(col0, DW)])
    pltpu.sync_copy(sbuf, outs_hbm.at[:, pl.ds(col0, DW)])

    @pl.when(w == 0)
    def _():
        lane = lax.iota(jnp.int32, L)
        for j in range(K // L):
            acc = zf16
            for r in range(L):
                row = cnt_acc[j * L + r, :] + nrows[j * L + r, :]
                acc = jnp.where(lane == r, row, acc)
            outcnt[pl.ds(j * L, L)] = acc.astype(jnp.int32)
        pltpu.sync_copy(outcnt, outn_hbm)


@jax.jit
def _run(centroids, states, keys, values, assign, cnt0):
    mesh = plsc.VectorSubcoreMesh(core_axis_name="c", subcore_axis_name="s")
    f = pl.kernel(
        _body,
        out_type=(
            jax.ShapeDtypeStruct((K, D), jnp.float32),
            jax.ShapeDtypeStruct((K, D), jnp.float32),
            jax.ShapeDtypeStruct((K,), jnp.int32),
        ),
        mesh=mesh,
        compiler_params=pltpu.CompilerParams(use_tc_tiling_on_sc=False),
        scratch_types=[
            pltpu.VMEM((K, DW), jnp.float32),
            pltpu.VMEM((K, DW), jnp.float32),
            pltpu.VMEM((K, L), jnp.float32),
            pltpu.VMEM((CHUNK, DW), jnp.float32),
            pltpu.VMEM((CHUNK, DW), jnp.float32),
            pltpu.VMEM((CHUNK,), jnp.int32),
            pltpu.VMEM((CHUNK, DW), jnp.float32),
            pltpu.VMEM((CHUNK, DW), jnp.float32),
            pltpu.VMEM((CHUNK,), jnp.int32),
            pltpu.VMEM((K, L), jnp.float32),
            pltpu.VMEM((K, DW), jnp.float32),
            pltpu.VMEM((K, DW), jnp.float32),
            pltpu.VMEM((K,), jnp.int32),
            pltpu.SemaphoreType.DMA,
            pltpu.SemaphoreType.DMA,
        ],
    )
    return f(centroids, states, keys, values, assign, cnt0)


def kernel(centroids, states, counts, keys, values, assign):
    assign = assign.astype(jnp.int32)
    cnt0 = jnp.broadcast_to(
        counts.astype(jnp.float32)[:, None], (K, L))
    return _run(centroids, states, keys, values, assign, cnt0)


# speedup vs baseline: 1.5826x; 1.1664x over previous
import jax
import jax.numpy as jnp
from jax import lax
from jax.experimental import pallas as pl
from jax.experimental.pallas import tpu as pltpu
from jax.experimental.pallas import tpu_sc as plsc

K = 64
D = 1024
N = 8192
NC = 2
NS = 16
NW = NC * NS
L = 16
DW = D // NW
CHUNK = 256
NCHUNK = N // CHUNK


def _body(cent_hbm, st_hbm, keys_hbm, vals_hbm, asg_hbm, cnt0_hbm,
          outc_hbm, outs_hbm, outn_hbm,
          acc_k, acc_v, cnt_acc,
          kbufa, vbufa, idxbufa, kbufb, vbufb, idxbufb,
          nrows, cbuf, sbuf, outcnt, sema, semb):
    cid = lax.axis_index("c")
    sid = lax.axis_index("s")
    w = cid * NS + sid
    col0 = w * DW

    zf16 = jnp.zeros((L,), jnp.float32)
    ones16 = jnp.ones((L,), jnp.float32)

    def _zf(r, _):
        acc_k[r, pl.ds(0, L)] = zf16
        acc_k[r, pl.ds(L, L)] = zf16
        acc_v[r, pl.ds(0, L)] = zf16
        acc_v[r, pl.ds(L, L)] = zf16
        cnt_acc[r, :] = zf16
        return 0
    lax.fori_loop(0, K, _zf, 0)

    def _start(nck, kb, vb, ib, sem):
        base = nck * CHUNK
        pltpu.async_copy(asg_hbm.at[pl.ds(base, CHUNK)], ib, sem)
        pltpu.async_copy(
            keys_hbm.at[pl.ds(base, CHUNK), pl.ds(col0, DW)], kb, sem)
        pltpu.async_copy(
            vals_hbm.at[pl.ds(base, CHUNK), pl.ds(col0, DW)], vb, sem)

    def _wait(kb, vb, ib, sem):
        pltpu.make_async_copy(asg_hbm.at[pl.ds(0, CHUNK)], ib, sem).wait()
        pltpu.make_async_copy(
            keys_hbm.at[pl.ds(0, CHUNK), pl.ds(0, DW)], kb, sem).wait()
        pltpu.make_async_copy(
            vals_hbm.at[pl.ds(0, CHUNK), pl.ds(0, DW)], vb, sem).wait()

    def _compute(kb, vb, ib):
        @plsc.parallel_loop(0, CHUNK // L, step=1, unroll=2)
        def _grp(q):
            iv = ib[pl.ds(q * L, L)]
            for t in range(L):
                a = iv[t]
                tok = q * L + t
                plsc.addupdate(acc_k.at[a, pl.ds(0, L)], kb[tok, pl.ds(0, L)])
                plsc.addupdate(acc_k.at[a, pl.ds(L, L)], kb[tok, pl.ds(L, L)])
                plsc.addupdate(acc_v.at[a, pl.ds(0, L)], vb[tok, pl.ds(0, L)])
                plsc.addupdate(acc_v.at[a, pl.ds(L, L)], vb[tok, pl.ds(L, L)])
                plsc.addupdate(cnt_acc.at[a], ones16)

    _start(0, kbufa, vbufa, idxbufa, sema)

    def _pair(gg, _):
        _wait(kbufa, vbufa, idxbufa, sema)
        _start(2 * gg + 1, kbufb, vbufb, idxbufb, semb)
        _compute(kbufa, vbufa, idxbufa)
        _wait(kbufb, vbufb, idxbufb, semb)

        @pl.when(gg < NCHUNK // 2 - 1)
        def _():
            _start(2 * gg + 2, kbufa, vbufa, idxbufa, sema)

        _compute(kbufb, vbufb, idxbufb)
        return 0

    lax.fori_loop(0, NCHUNK // 2, _pair, 0)

    pltpu.sync_copy(cnt0_hbm, nrows)
    pltpu.sync_copy(cent_hbm.at[:, pl.ds(col0, DW)], cbuf)
    pltpu.sync_copy(st_hbm.at[:, pl.ds(col0, DW)], sbuf)

    def _mg(r, _):
        nvec = nrows[r, :]
        mvec = cnt_acc[r, :]
        denom = nvec + mvec
        pos = denom > 0.5
        inv = 1.0 / jnp.where(pos, denom, 1.0)
        for j in range(DW // L):
            sl = pl.ds(j * L, L)
            c = cbuf[r, sl]
            s = sbuf[r, sl]
            cbuf[r, sl] = jnp.where(pos, (nvec * c + acc_k[r, sl]) * inv, c)
            sbuf[r, sl] = jnp.where(pos, (nvec * s + acc_v[r, sl]) * inv, s)
        return 0

    lax.fori_loop(0, K, _mg, 0)
    pltpu.sync_copy(cbuf, outc_hbm.at[:, pl.ds(col0, DW)])
    pltpu.sync_copy(sbuf, outs_hbm.at[:, pl.ds(col0, DW)])

    @pl.when(w == 0)
    def _():
        lane = lax.iota(jnp.int32, L)
        for j in range(K // L):
            acc = zf16
            for r in range(L):
                row = cnt_acc[j * L + r, :] + nrows[j * L + r, :]
                acc = jnp.where(lane == r, row, acc)
            outcnt[pl.ds(j * L, L)] = acc.astype(jnp.int32)
        pltpu.sync_copy(outcnt, outn_hbm)


@jax.jit
def _run(centroids, states, keys, values, assign, cnt0):
    mesh = plsc.VectorSubcoreMesh(core_axis_name="c", subcore_axis_name="s")
    f = pl.kernel(
        _body,
        out_type=(
            jax.ShapeDtypeStruct((K, D), jnp.float32),
            jax.ShapeDtypeStruct((K, D), jnp.float32),
            jax.ShapeDtypeStruct((K,), jnp.int32),
        ),
        mesh=mesh,
        compiler_params=pltpu.CompilerParams(use_tc_tiling_on_sc=False),
        scratch_types=[
            pltpu.VMEM((K, DW), jnp.float32),
            pltpu.VMEM((K, DW), jnp.float32),
            pltpu.VMEM((K, L), jnp.float32),
            pltpu.VMEM((CHUNK, DW), jnp.float32),
            pltpu.VMEM((CHUNK, DW), jnp.float32),
            pltpu.VMEM((CHUNK,), jnp.int32),
            pltpu.VMEM((CHUNK, DW), jnp.float32),
            pltpu.VMEM((CHUNK, DW), jnp.float32),
            pltpu.VMEM((CHUNK,), jnp.int32),
            pltpu.VMEM((K, L), jnp.float32),
            pltpu.VMEM((K, DW), jnp.float32),
            pltpu.VMEM((K, DW), jnp.float32),
            pltpu.VMEM((K,), jnp.int32),
            pltpu.SemaphoreType.DMA,
            pltpu.SemaphoreType.DMA,
        ],
    )
    return f(centroids, states, keys, values, assign, cnt0)


def kernel(centroids, states, counts, keys, values, assign):
    assign = assign.astype(jnp.int32)
    cnt0 = jnp.broadcast_to(
        counts.astype(jnp.float32)[:, None], (K, L))
    return _run(centroids, states, keys, values, assign, cnt0)


# speedup vs baseline: 1.9067x; 1.2048x over previous
import jax
import jax.numpy as jnp
from jax import lax
from jax.experimental import pallas as pl
from jax.experimental.pallas import tpu as pltpu
from jax.experimental.pallas import tpu_sc as plsc

K = 64
D = 1024
N = 8192
NC = 2
NS = 16
L = 16
NG = 4
NT = 4
DG = 128
DH = NG * DG
TPG = N // NT
RPT = K // NT
CH = 128
NCH = TPG // CH


def _body(cent_hbm, st_hbm, keys_hbm, vals_hbm, asg_hbm, cnt0_hbm,
          outc_hbm, outs_hbm, outn_hbm,
          stage_k, stage_v, stage_c,
          acc_k, acc_v, cnt_acc,
          kbufa, vbufa, idxa, kbufb, vbufb, idxb,
          cbuf, sbuf, skbuf, svbuf, tbuf, nr, mr, ctmp, outcnt,
          sina, sinb):
    cid = lax.axis_index("c")
    sid = lax.axis_index("s")
    gl = sid // NT
    t = sid % NT
    gcol = cid * DH + gl * DG
    tok0 = t * TPG
    r0 = t * RPT

    zf16 = jnp.zeros((L,), jnp.float32)
    ones16 = jnp.ones((L,), jnp.float32)

    def _zf(r, _):
        for j in range(DG // L):
            acc_k[r, pl.ds(j * L, L)] = zf16
            acc_v[r, pl.ds(j * L, L)] = zf16
        cnt_acc[r, pl.ds(0, L)] = zf16
        return 0
    lax.fori_loop(0, K, _zf, 0)

    def _start_in(c, kb, vb, ib, sem):
        base = tok0 + c * CH
        pltpu.async_copy(asg_hbm.at[pl.ds(base, CH)], ib, sem)
        pltpu.async_copy(
            keys_hbm.at[pl.ds(base, CH), pl.ds(gcol, DG)], kb, sem)
        pltpu.async_copy(
            vals_hbm.at[pl.ds(base, CH), pl.ds(gcol, DG)], vb, sem)

    def _wait_in(kb, vb, ib, sem):
        pltpu.make_async_copy(asg_hbm.at[pl.ds(0, CH)], ib, sem).wait()
        pltpu.make_async_copy(
            keys_hbm.at[pl.ds(0, CH), pl.ds(0, DG)], kb, sem).wait()
        pltpu.make_async_copy(
            vals_hbm.at[pl.ds(0, CH), pl.ds(0, DG)], vb, sem).wait()

    def _compute(kb, vb, ib):
        @plsc.parallel_loop(0, CH // L, step=1, unroll=2)
        def _grp(q):
            iv = ib[pl.ds(q * L, L)]
            for tt in range(L):
                a = iv[tt]
                tok = q * L + tt
                for j in range(DG // L):
                    sl = pl.ds(j * L, L)
                    plsc.addupdate(acc_k.at[a, sl], kb[tok, sl])
                for j in range(DG // L):
                    sl = pl.ds(j * L, L)
                    plsc.addupdate(acc_v.at[a, sl], vb[tok, sl])

        @pl.when(gl == 0)
        def _():
            @plsc.parallel_loop(0, CH // L, step=1, unroll=2)
            def _cgrp(q):
                iv = ib[pl.ds(q * L, L)]
                for tt in range(L):
                    a = iv[tt]
                    plsc.addupdate(cnt_acc.at[a, pl.ds(0, L)], ones16)

    _start_in(0, kbufa, vbufa, idxa, sina)

    def _pair(p, _):
        _wait_in(kbufa, vbufa, idxa, sina)
        _start_in(2 * p + 1, kbufb, vbufb, idxb, sinb)
        _compute(kbufa, vbufa, idxa)
        _wait_in(kbufb, vbufb, idxb, sinb)

        @pl.when(p < NCH // 2 - 1)
        def _():
            _start_in(2 * p + 2, kbufa, vbufa, idxa, sina)

        _compute(kbufb, vbufb, idxb)
        return 0

    lax.fori_loop(0, NCH // 2, _pair, 0)

    pltpu.sync_copy(acc_k, stage_k.at[sid])
    pltpu.sync_copy(acc_v, stage_v.at[sid])

    @pl.when(gl == 0)
    def _():
        pltpu.sync_copy(cnt_acc, stage_c.at[t])

    plsc.subcore_barrier()

    pltpu.sync_copy(stage_k.at[gl * NT].at[pl.ds(r0, RPT)], skbuf)
    pltpu.sync_copy(stage_v.at[gl * NT].at[pl.ds(r0, RPT)], svbuf)
    pltpu.sync_copy(stage_c.at[0].at[pl.ds(r0, RPT)], mr)
    for t2 in range(1, NT):
        pltpu.sync_copy(stage_k.at[gl * NT + t2].at[pl.ds(r0, RPT)], tbuf)

        def _addk(r, _):
            for j in range(DG // L):
                sl = pl.ds(j * L, L)
                skbuf[r, sl] = skbuf[r, sl] + tbuf[r, sl]
            return 0
        lax.fori_loop(0, RPT, _addk, 0)
        pltpu.sync_copy(stage_v.at[gl * NT + t2].at[pl.ds(r0, RPT)], tbuf)

        def _addv(r, _):
            for j in range(DG // L):
                sl = pl.ds(j * L, L)
                svbuf[r, sl] = svbuf[r, sl] + tbuf[r, sl]
            return 0
        lax.fori_loop(0, RPT, _addv, 0)
        pltpu.sync_copy(stage_c.at[t2].at[pl.ds(r0, RPT)], ctmp)

        def _addc(r, _):
            sl = pl.ds(0, L)
            mr[r, sl] = mr[r, sl] + ctmp[r, sl]
            return 0
        lax.fori_loop(0, RPT, _addc, 0)

    pltpu.sync_copy(cnt0_hbm.at[pl.ds(r0, RPT)], nr)
    pltpu.sync_copy(cent_hbm.at[pl.ds(r0, RPT), pl.ds(gcol, DG)], cbuf)
    pltpu.sync_copy(st_hbm.at[pl.ds(r0, RPT), pl.ds(gcol, DG)], sbuf)

    def _mg(r, _):
        nvec = nr[r, pl.ds(0, L)]
        mvec = mr[r, pl.ds(0, L)]
        denom = nvec + mvec
        pos = denom > 0.5
        inv = 1.0 / jnp.where(pos, denom, 1.0)
        for j in range(DG // L):
            sl = pl.ds(j * L, L)
            c = cbuf[r, sl]
            s = sbuf[r, sl]
            cbuf[r, sl] = jnp.where(pos, (nvec * c + skbuf[r, sl]) * inv, c)
            sbuf[r, sl] = jnp.where(pos, (nvec * s + svbuf[r, sl]) * inv, s)
        return 0

    lax.fori_loop(0, RPT, _mg, 0)
    pltpu.sync_copy(cbuf, outc_hbm.at[pl.ds(r0, RPT), pl.ds(gcol, DG)])
    pltpu.sync_copy(sbuf, outs_hbm.at[pl.ds(r0, RPT), pl.ds(gcol, DG)])

    @pl.when((cid == 0) & (gl == 0))
    def _():
        lane = lax.iota(jnp.int32, L)
        acc = zf16
        for r in range(RPT):
            row = mr[r, pl.ds(0, L)] + nr[r, pl.ds(0, L)]
            acc = jnp.where(lane == r, row, acc)
        outcnt[:] = acc.astype(jnp.int32)
        pltpu.sync_copy(outcnt, outn_hbm.at[pl.ds(r0, RPT)])


@jax.jit
def _run(centroids, states, keys, values, assign, cnt0):
    mesh = plsc.VectorSubcoreMesh(core_axis_name="c", subcore_axis_name="s")
    f = pl.kernel(
        _body,
        out_type=(
            jax.ShapeDtypeStruct((K, D), jnp.float32),
            jax.ShapeDtypeStruct((K, D), jnp.float32),
            jax.ShapeDtypeStruct((K,), jnp.int32),
        ),
        mesh=mesh,
        scratch_types=[
            pltpu.VMEM_SHARED((NS, K, DG), jnp.float32),
            pltpu.VMEM_SHARED((NS, K, DG), jnp.float32),
            pltpu.VMEM_SHARED((NT, K, DG), jnp.float32),
            pltpu.VMEM((K, DG), jnp.float32),
            pltpu.VMEM((K, DG), jnp.float32),
            pltpu.VMEM((K, DG), jnp.float32),
            pltpu.VMEM((CH, DG), jnp.float32),
            pltpu.VMEM((CH, DG), jnp.float32),
            pltpu.VMEM((CH,), jnp.int32),
            pltpu.VMEM((CH, DG), jnp.float32),
            pltpu.VMEM((CH, DG), jnp.float32),
            pltpu.VMEM((CH,), jnp.int32),
            pltpu.VMEM((RPT, DG), jnp.float32),
            pltpu.VMEM((RPT, DG), jnp.float32),
            pltpu.VMEM((RPT, DG), jnp.float32),
            pltpu.VMEM((RPT, DG), jnp.float32),
            pltpu.VMEM((RPT, DG), jnp.float32),
            pltpu.VMEM((RPT, DG), jnp.float32),
            pltpu.VMEM((RPT, DG), jnp.float32),
            pltpu.VMEM((RPT, DG), jnp.float32),
            pltpu.VMEM((RPT,), jnp.int32),
            pltpu.SemaphoreType.DMA,
            pltpu.SemaphoreType.DMA,
        ],
    )
    return f(centroids, states, keys, values, assign, cnt0)


def kernel(centroids, states, counts, keys, values, assign):
    assign = assign.astype(jnp.int32)
    cnt0 = jnp.broadcast_to(
        counts.astype(jnp.float32)[:, None], (K, DG))
    return _run(centroids, states, keys, values, assign, cnt0)


# speedup vs baseline: 3.4038x; 1.7852x over previous
import jax
import jax.numpy as jnp
from jax import lax
from jax.experimental import pallas as pl
from jax.experimental.pallas import tpu as pltpu
from jax.experimental.pallas import tpu_sc as plsc

K = 64
D = 1024
N = 8192
NC = 2
NS = 16
L = 16
NG = 4
NT = 4
DG = 128
DH = NG * DG
TPG = N // NT
RPT = K // NT
CH = 128
NCH = TPG // CH


def _body(cent_hbm, st_hbm, keys_hbm, vals_hbm, asg_hbm, cnt0_hbm,
          outc_hbm, outs_hbm, outn_hbm,
          stage_k, stage_v, stage_c,
          acc_k, acc_v, cnt_acc,
          kbufa, vbufa, idxa, kbufb, vbufb, idxb,
          cbuf, sbuf, skbuf, svbuf, tbuf, nr, mr, ctmp, outcnt,
          sina, sinb):
    cid = lax.axis_index("c")
    sid = lax.axis_index("s")
    gl = sid // NT
    t = sid % NT
    gcol = cid * DH + gl * DG
    tok0 = t * TPG
    r0 = t * RPT

    zf16 = jnp.zeros((L,), jnp.float32)
    ones16 = jnp.ones((L,), jnp.float32)

    def _zf(r, _):
        for j in range(DG // L):
            acc_k[r, pl.ds(j * L, L)] = zf16
            acc_v[r, pl.ds(j * L, L)] = zf16
        cnt_acc[r, pl.ds(0, L)] = zf16
        return 0
    lax.fori_loop(0, K, _zf, 0)

    def _start_in(c, kb, vb, ib, sem):
        base = tok0 + c * CH
        pltpu.async_copy(asg_hbm.at[pl.ds(base, CH)], ib, sem)
        pltpu.async_copy(
            keys_hbm.at[pl.ds(base, CH), pl.ds(gcol, DG)], kb, sem)
        pltpu.async_copy(
            vals_hbm.at[pl.ds(base, CH), pl.ds(gcol, DG)], vb, sem)

    def _wait_in(kb, vb, ib, sem):
        pltpu.make_async_copy(asg_hbm.at[pl.ds(0, CH)], ib, sem).wait()
        pltpu.make_async_copy(
            keys_hbm.at[pl.ds(0, CH), pl.ds(0, DG)], kb, sem).wait()
        pltpu.make_async_copy(
            vals_hbm.at[pl.ds(0, CH), pl.ds(0, DG)], vb, sem).wait()

    def _compute(kb, vb, ib):
        @plsc.parallel_loop(0, CH // L, step=1, unroll=2)
        def _grp(q):
            iv = ib[pl.ds(q * L, L)]

            def _loads(tok):
                return (
                    [kb[tok, pl.ds(j * L, L)] for j in range(DG // L)],
                    [vb[tok, pl.ds(j * L, L)] for j in range(DG // L)],
                )

            cur = _loads(q * L)
            acur = iv[0]
            for tt in range(L):
                if tt + 1 < L:
                    nxt = _loads(q * L + tt + 1)
                    anxt = iv[tt + 1]
                kcur, vcur = cur
                for j in range(DG // L):
                    plsc.addupdate(acc_k.at[acur, pl.ds(j * L, L)], kcur[j])
                for j in range(DG // L):
                    plsc.addupdate(acc_v.at[acur, pl.ds(j * L, L)], vcur[j])
                if tt + 1 < L:
                    cur = nxt
                    acur = anxt

        @pl.when(gl == 0)
        def _():
            @plsc.parallel_loop(0, CH // L, step=1, unroll=2)
            def _cgrp(q):
                iv = ib[pl.ds(q * L, L)]
                for tt in range(L):
                    a = iv[tt]
                    plsc.addupdate(cnt_acc.at[a, pl.ds(0, L)], ones16)

    _start_in(0, kbufa, vbufa, idxa, sina)

    def _pair(p, _):
        _wait_in(kbufa, vbufa, idxa, sina)
        _start_in(2 * p + 1, kbufb, vbufb, idxb, sinb)
        _compute(kbufa, vbufa, idxa)
        _wait_in(kbufb, vbufb, idxb, sinb)

        @pl.when(p < NCH // 2 - 1)
        def _():
            _start_in(2 * p + 2, kbufa, vbufa, idxa, sina)

        _compute(kbufb, vbufb, idxb)
        return 0

    lax.fori_loop(0, NCH // 2, _pair, 0)

    pltpu.sync_copy(acc_k, stage_k.at[sid])
    pltpu.sync_copy(acc_v, stage_v.at[sid])

    @pl.when(gl == 0)
    def _():
        pltpu.sync_copy(cnt_acc, stage_c.at[t])

    plsc.subcore_barrier()

    pltpu.sync_copy(stage_k.at[gl * NT].at[pl.ds(r0, RPT)], skbuf)
    pltpu.sync_copy(stage_v.at[gl * NT].at[pl.ds(r0, RPT)], svbuf)
    pltpu.sync_copy(stage_c.at[0].at[pl.ds(r0, RPT)], mr)
    for t2 in range(1, NT):
        pltpu.sync_copy(stage_k.at[gl * NT + t2].at[pl.ds(r0, RPT)], tbuf)

        def _addk(r, _):
            for j in range(DG // L):
                sl = pl.ds(j * L, L)
                skbuf[r, sl] = skbuf[r, sl] + tbuf[r, sl]
            return 0
        lax.fori_loop(0, RPT, _addk, 0)
        pltpu.sync_copy(stage_v.at[gl * NT + t2].at[pl.ds(r0, RPT)], tbuf)

        def _addv(r, _):
            for j in range(DG // L):
                sl = pl.ds(j * L, L)
                svbuf[r, sl] = svbuf[r, sl] + tbuf[r, sl]
            return 0
        lax.fori_loop(0, RPT, _addv, 0)
        pltpu.sync_copy(stage_c.at[t2].at[pl.ds(r0, RPT)], ctmp)

        def _addc(r, _):
            sl = pl.ds(0, L)
            mr[r, sl] = mr[r, sl] + ctmp[r, sl]
            return 0
        lax.fori_loop(0, RPT, _addc, 0)

    pltpu.sync_copy(cnt0_hbm.at[pl.ds(r0, RPT)], nr)
    pltpu.sync_copy(cent_hbm.at[pl.ds(r0, RPT), pl.ds(gcol, DG)], cbuf)
    pltpu.sync_copy(st_hbm.at[pl.ds(r0, RPT), pl.ds(gcol, DG)], sbuf)

    def _mg(r, _):
        nvec = nr[r, pl.ds(0, L)]
        mvec = mr[r, pl.ds(0, L)]
        denom = nvec + mvec
        pos = denom > 0.5
        inv = 1.0 / jnp.where(pos, denom, 1.0)
        for j in range(DG // L):
            sl = pl.ds(j * L, L)
            c = cbuf[r, sl]
            s = sbuf[r, sl]
            cbuf[r, sl] = jnp.where(pos, (nvec * c + skbuf[r, sl]) * inv, c)
            sbuf[r, sl] = jnp.where(pos, (nvec * s + svbuf[r, sl]) * inv, s)
        return 0

    lax.fori_loop(0, RPT, _mg, 0)
    pltpu.sync_copy(cbuf, outc_hbm.at[pl.ds(r0, RPT), pl.ds(gcol, DG)])
    pltpu.sync_copy(sbuf, outs_hbm.at[pl.ds(r0, RPT), pl.ds(gcol, DG)])

    @pl.when((cid == 0) & (gl == 0))
    def _():
        lane = lax.iota(jnp.int32, L)
        acc = zf16
        for r in range(RPT):
            row = mr[r, pl.ds(0, L)] + nr[r, pl.ds(0, L)]
            acc = jnp.where(lane == r, row, acc)
        outcnt[:] = acc.astype(jnp.int32)
        pltpu.sync_copy(outcnt, outn_hbm.at[pl.ds(r0, RPT)])


@jax.jit
def _run(centroids, states, keys, values, assign, cnt0):
    mesh = plsc.VectorSubcoreMesh(core_axis_name="c", subcore_axis_name="s")
    f = pl.kernel(
        _body,
        out_type=(
            jax.ShapeDtypeStruct((K, D), jnp.float32),
            jax.ShapeDtypeStruct((K, D), jnp.float32),
            jax.ShapeDtypeStruct((K,), jnp.int32),
        ),
        mesh=mesh,
        scratch_types=[
            pltpu.VMEM_SHARED((NS, K, DG), jnp.float32),
            pltpu.VMEM_SHARED((NS, K, DG), jnp.float32),
            pltpu.VMEM_SHARED((NT, K, DG), jnp.float32),
            pltpu.VMEM((K, DG), jnp.float32),
            pltpu.VMEM((K, DG), jnp.float32),
            pltpu.VMEM((K, DG), jnp.float32),
            pltpu.VMEM((CH, DG), jnp.float32),
            pltpu.VMEM((CH, DG), jnp.float32),
            pltpu.VMEM((CH,), jnp.int32),
            pltpu.VMEM((CH, DG), jnp.float32),
            pltpu.VMEM((CH, DG), jnp.float32),
            pltpu.VMEM((CH,), jnp.int32),
            pltpu.VMEM((RPT, DG), jnp.float32),
            pltpu.VMEM((RPT, DG), jnp.float32),
            pltpu.VMEM((RPT, DG), jnp.float32),
            pltpu.VMEM((RPT, DG), jnp.float32),
            pltpu.VMEM((RPT, DG), jnp.float32),
            pltpu.VMEM((RPT, DG), jnp.float32),
            pltpu.VMEM((RPT, DG), jnp.float32),
            pltpu.VMEM((RPT, DG), jnp.float32),
            pltpu.VMEM((RPT,), jnp.int32),
            pltpu.SemaphoreType.DMA,
            pltpu.SemaphoreType.DMA,
        ],
    )
    return f(centroids, states, keys, values, assign, cnt0)


def kernel(centroids, states, counts, keys, values, assign):
    assign = assign.astype(jnp.int32)
    cnt0 = jnp.broadcast_to(
        counts.astype(jnp.float32)[:, None], (K, DG))
    return _run(centroids, states, keys, values, assign, cnt0)
